# baseline (speedup 1.0000x reference)
import functools
import math

import jax
import jax.numpy as jnp
from jax import lax
from jax.experimental import pallas as pl
from jax.experimental.pallas import tpu as pltpu
from jax.experimental.pallas import tpu_sc as plsc

D_EMBED = 64
SCALE = math.sqrt(D_EMBED)
NUM_CORES = 2
NUM_SUBCORES = 16
NUM_WORKERS = NUM_CORES * NUM_SUBCORES
LANES = 16
NBUF = 4
LOOKAHEAD = 2
PREP_W = 1024
HALF = PREP_W // 2


def _prep_block(tab_t_ref, out_ref):
    block = tab_t_ref[...]
    lo = block[:, :HALF].T * SCALE
    hi = block[:, HALF:].T * SCALE
    out_ref[...] = jnp.concatenate([lo, hi], axis=1)


def _prep_table(tab_t):
    v = tab_t.shape[1]
    n_blocks = (v + PREP_W - 1) // PREP_W
    return pl.pallas_call(
        _prep_block,
        grid=(n_blocks,),
        in_specs=[pl.BlockSpec((D_EMBED, PREP_W), lambda i: (0, i))],
        out_specs=pl.BlockSpec((HALF, 2 * D_EMBED), lambda i: (i, 0)),
        out_shape=jax.ShapeDtypeStruct((n_blocks * HALF, 2 * D_EMBED), jnp.float32),
    )(tab_t)


def _build_sc_gather(xb: int, xt: int, v_pad: int):
    assert xb % (NUM_WORKERS * NBUF) == 0
    rows_per_worker = xb // NUM_WORKERS

    mesh = plsc.VectorSubcoreMesh(core_axis_name="c", subcore_axis_name="s")

    @functools.partial(
        pl.kernel,
        out_type=jax.ShapeDtypeStruct((xb, xt, D_EMBED), jnp.float32),
        mesh=mesh,
        scratch_types=[
            pltpu.VMEM((rows_per_worker, xt), jnp.int32),
            pltpu.VMEM((rows_per_worker, xt), jnp.int32),
            pltpu.VMEM((NBUF, xt, D_EMBED), jnp.float32),
            pltpu.SemaphoreType.DMA((NBUF,)),
            pltpu.SemaphoreType.DMA((NBUF,)),
        ],
        compiler_params=pltpu.CompilerParams(use_tc_tiling_on_sc=False),
    )
    def sc_gather(x_hbm, tab_hbm, out_hbm, idx_v, idxq_v, rows_v, gsem, ssem):
        wid = lax.axis_index("s") * NUM_CORES + lax.axis_index("c")
        base = wid * rows_per_worker
        pltpu.sync_copy(x_hbm.at[pl.ds(base, rows_per_worker)], idx_v)

        col_starts = [c * LANES for c in range(xt // LANES)]
        if xt % LANES:
            col_starts.append(xt - LANES)

        @pl.loop(0, rows_per_worker)
        def _remap(r):
            for c0 in col_starts:
                sl = pl.ds(c0, LANES)
                i = idx_v[r, sl]
                q = (i & ~(PREP_W - 1)) + ((i & (HALF - 1)) << 1) + (
                    (i >> 9) & 1
                )
                idxq_v[r, sl] = q

        def gather(r, b):
            return pltpu.make_async_copy(
                tab_hbm.at[idxq_v.at[r]],
                rows_v.at[b],
                gsem.at[b],
            )

        def store(r, b):
            return pltpu.make_async_copy(
                rows_v.at[b],
                out_hbm.at[base + r],
                ssem.at[b],
            )

        for r in range(LOOKAHEAD):
            gather(r, r).start()

        @pl.loop(0, rows_per_worker // NBUF)
        def _group(g):
            r0 = g * NBUF
            for b in range(NBUF):
                r = r0 + b
                b2 = (b + LOOKAHEAD) % NBUF

                gather(r, b).wait()

                @pl.when(r + LOOKAHEAD < rows_per_worker)
                def _start_next():
                    @pl.when(r + LOOKAHEAD >= NBUF)
                    def _drain_b2():
                        store(0, b2).wait()

                    gather(r + LOOKAHEAD, b2).start()

                store(r, b).start()

        for b in range(NBUF):
            store(0, b).wait()

    return sc_gather


def kernel(x, table):
    b, t = x.shape
    scaled2 = _prep_table(table.T)
    v_pad = scaled2.shape[0] * 2
    scaled = scaled2.reshape(v_pad, D_EMBED)
    return _build_sc_gather(b, t, v_pad)(x.astype(jnp.int32), scaled)
